# baseline (speedup 1.0000x reference)
import functools

import jax
import jax.numpy as jnp
from jax.experimental import pallas as pl
from jax.experimental.pallas import tpu as pltpu

H_IMG = 96
W_IMG = 96
Q = H_IMG * W_IMG
NSUB = 8
M = Q // NSUB
K = 9
PAD = 128
QT = 384
NT = Q // QT


def _layer_kernel(tpad_ref, cand_ref, wmat_ref, kw_ref, lw_ref, bias_ref,
                  out_ref, *, cout, relu):
    f32 = jnp.float32
    t = pl.program_id(1)
    q0 = t * QT

    widx = jax.lax.broadcasted_iota(jnp.int32, (QT, 1), 0) % W_IMG
    acc = jnp.zeros((QT, cout), f32)
    for j in range(9):
        dy = j // 3 - 1
        dx = j % 3 - 1
        off = PAD + q0 + dy * W_IMG + dx
        xs = tpad_ref[0, pl.ds(off, QT), :]
        if dx == -1:
            xs = jnp.where(widx > 0, xs, 0.0)
        elif dx == 1:
            xs = jnp.where(widx < W_IMG - 1, xs, 0.0)
        acc = acc + jax.lax.dot_general(
            xs, wmat_ref[j], (((1,), (0,)), ((), ())),
            preferred_element_type=f32)

    tok = tpad_ref[0, pl.ds(PAD + q0, QT), :]
    cand = cand_ref[0]
    q2 = jnp.sum(tok * tok, axis=1, keepdims=True)
    c2 = jnp.sum(cand * cand, axis=1)[None, :]
    dist = q2 - 2.0 * jax.lax.dot_general(
        tok, cand, (((1,), (1,)), ((), ())),
        preferred_element_type=f32) + c2

    d = dist
    it = jax.lax.broadcasted_iota(jnp.int32, (QT, M), 1)
    it_f = it.astype(f32)
    BIGD = f32(3e38)
    v = jnp.full((QT, 1), -BIGD, f32)
    mi = jnp.full((QT, 1), jnp.int32(-1))
    amins = []
    for k in range(K):
        elig = (d > v) | ((d == v) & (it > mi))
        dmin = jnp.min(jnp.where(elig, d, BIGD), axis=1, keepdims=True)
        amin = jnp.min(jnp.where(elig & (d == dmin), it_f, f32(4096.0)),
                       axis=1, keepdims=True)
        amins.append(amin)
        v = dmin
        mi = amin.astype(jnp.int32)
    wsel = jnp.zeros((QT, M), f32)
    for k in range(K):
        wsel = jnp.where(it_f == amins[k], kw_ref[0:1, k:k + 1], wsel)

    agg = jax.lax.dot_general(wsel, cand, (((1,), (0,)), ((), ())),
                              preferred_element_type=f32)
    y = acc + jax.lax.dot_general(agg, lw_ref[...], (((1,), (0,)), ((), ())),
                                  preferred_element_type=f32) + bias_ref[...]
    if relu:
        y = jnp.maximum(y, 0.0)
    out_ref[0] = y


def _layer(tokens, cw, cb, kw, lw, lb, relu, *, interpret=False):
    B, _, cin = tokens.shape
    cout = lw.shape[1]
    tpad = jnp.pad(tokens, ((0, 0), (PAD, PAD), (0, 0)))
    cand = tokens[:, ::NSUB, :]
    wmat = cw.transpose(2, 3, 1, 0).reshape(9, cin, cout)
    bias = (cb + lb)[None, :]
    kw2 = kw[None, :]
    out = pl.pallas_call(
        functools.partial(_layer_kernel, cout=cout, relu=relu),
        grid=(B, NT),
        in_specs=[
            pl.BlockSpec((1, Q + 2 * PAD, cin), lambda b, t: (b, 0, 0)),
            pl.BlockSpec((1, M, cin), lambda b, t: (b, 0, 0)),
            pl.BlockSpec((9, cin, cout), lambda b, t: (0, 0, 0)),
            pl.BlockSpec((1, K), lambda b, t: (0, 0)),
            pl.BlockSpec((cin, cout), lambda b, t: (0, 0)),
            pl.BlockSpec((1, cout), lambda b, t: (0, 0)),
        ],
        out_specs=pl.BlockSpec((1, QT, cout), lambda b, t: (b, t, 0)),
        out_shape=jax.ShapeDtypeStruct((B, Q, cout), jnp.float32),
        compiler_params=pltpu.CompilerParams(
            dimension_semantics=("parallel", "parallel")),
        interpret=interpret,
    )(tpad, cand, wmat, kw2, lw, bias)
    return out


def kernel(x, conv1_w, conv1_b, knn1_k, knn1_w, knn1_b,
           conv2_w, conv2_b, knn2_k, knn2_w, knn2_b,
           conv3_w, conv3_b, knn3_k, knn3_w, knn3_b,
           interpret=False):
    B = x.shape[0]
    tokens = x.reshape(B, -1, Q).transpose(0, 2, 1)
    t1 = _layer(tokens, conv1_w, conv1_b, knn1_k, knn1_w, knn1_b, True,
                interpret=interpret)
    t2 = _layer(t1, conv2_w, conv2_b, knn2_k, knn2_w, knn2_b, True,
                interpret=interpret)
    t3 = _layer(t2, conv3_w, conv3_b, knn3_k, knn3_w, knn3_b, False,
                interpret=interpret)
    return t3.transpose(0, 2, 1).reshape(B, -1, H_IMG, W_IMG)


# speedup vs baseline: 1.8187x; 1.8187x over previous
import functools

import jax
import jax.numpy as jnp
from jax.experimental import pallas as pl
from jax.experimental.pallas import tpu as pltpu

H_IMG = 96
W_IMG = 96
Q = H_IMG * W_IMG
NSUB = 8
M = Q // NSUB
K = 9
PAD = 128
QT = 384
NT = Q // QT


def _layer_kernel(tpad_ref, cand_ref, wmat_ref, kw_ref, lw_ref, bias_ref,
                  out_ref, *, cout, relu):
    f32 = jnp.float32
    t = pl.program_id(1)
    q0 = t * QT

    widx = jax.lax.broadcasted_iota(jnp.int32, (QT, 1), 0) % W_IMG
    acc = jnp.zeros((QT, cout), f32)
    for j in range(9):
        dy = j // 3 - 1
        dx = j % 3 - 1
        off = PAD + q0 + dy * W_IMG + dx
        xs = tpad_ref[0, pl.ds(off, QT), :]
        if dx == -1:
            xs = jnp.where(widx > 0, xs, 0.0)
        elif dx == 1:
            xs = jnp.where(widx < W_IMG - 1, xs, 0.0)
        acc = acc + jax.lax.dot_general(
            xs, wmat_ref[j], (((1,), (0,)), ((), ())),
            preferred_element_type=f32)

    tok = tpad_ref[0, pl.ds(PAD + q0, QT), :]
    cand = cand_ref[0]
    q2 = jnp.sum(tok * tok, axis=1, keepdims=True)
    c2 = jnp.sum(cand * cand, axis=1)[None, :]
    dist = q2 - 2.0 * jax.lax.dot_general(
        tok, cand, (((1,), (1,)), ((), ())),
        preferred_element_type=f32) + c2

    it = jax.lax.broadcasted_iota(jnp.int32, (QT, M), 1)
    bits = jax.lax.bitcast_convert_type(jnp.maximum(dist, 0.0), jnp.int32)
    bits = ((bits + 0x8) & ~0xF | (it >> 7)) + 0x00800000
    key = jax.lax.bitcast_convert_type(bits, f32)
    io128 = jax.lax.broadcasted_iota(jnp.int32, (QT, 128), 1).astype(f32)
    BIGD = f32(3e38)
    wsel = jnp.zeros((QT, M), f32)
    NG = M // 128
    for k in range(K):
        mval = key[:, :128]
        for g in range(1, NG):
            mval = jnp.minimum(mval, key[:, g * 128:(g + 1) * 128])
        kmin = jnp.min(mval, axis=1, keepdims=True)
        lstar = jnp.min(jnp.where(mval == kmin, io128, f32(256.0)),
                        axis=1, keepdims=True)
        gstar = jax.lax.bitcast_convert_type(kmin, jnp.int32) & 0xF
        mstar = gstar * 128 + lstar.astype(jnp.int32)
        ohm = it == mstar
        wsel = jnp.where(ohm, kw_ref[0:1, k:k + 1], wsel)
        if k < K - 1:
            key = jnp.where(ohm, BIGD, key)

    agg = jax.lax.dot_general(wsel, cand, (((1,), (0,)), ((), ())),
                              preferred_element_type=f32)
    y = acc + jax.lax.dot_general(agg, lw_ref[...], (((1,), (0,)), ((), ())),
                                  preferred_element_type=f32) + bias_ref[...]
    if relu:
        y = jnp.maximum(y, 0.0)
    out_ref[0] = y


def _layer(tokens, cw, cb, kw, lw, lb, relu, *, interpret=False):
    B, _, cin = tokens.shape
    cout = lw.shape[1]
    tpad = jnp.pad(tokens, ((0, 0), (PAD, PAD), (0, 0)))
    cand = tokens[:, ::NSUB, :]
    wmat = cw.transpose(2, 3, 1, 0).reshape(9, cin, cout)
    bias = (cb + lb)[None, :]
    kw2 = kw[None, :]
    out = pl.pallas_call(
        functools.partial(_layer_kernel, cout=cout, relu=relu),
        grid=(B, NT),
        in_specs=[
            pl.BlockSpec((1, Q + 2 * PAD, cin), lambda b, t: (b, 0, 0)),
            pl.BlockSpec((1, M, cin), lambda b, t: (b, 0, 0)),
            pl.BlockSpec((9, cin, cout), lambda b, t: (0, 0, 0)),
            pl.BlockSpec((1, K), lambda b, t: (0, 0)),
            pl.BlockSpec((cin, cout), lambda b, t: (0, 0)),
            pl.BlockSpec((1, cout), lambda b, t: (0, 0)),
        ],
        out_specs=pl.BlockSpec((1, QT, cout), lambda b, t: (b, t, 0)),
        out_shape=jax.ShapeDtypeStruct((B, Q, cout), jnp.float32),
        compiler_params=pltpu.CompilerParams(
            dimension_semantics=("parallel", "parallel")),
        interpret=interpret,
    )(tpad, cand, wmat, kw2, lw, bias)
    return out


def kernel(x, conv1_w, conv1_b, knn1_k, knn1_w, knn1_b,
           conv2_w, conv2_b, knn2_k, knn2_w, knn2_b,
           conv3_w, conv3_b, knn3_k, knn3_w, knn3_b,
           interpret=False):
    B = x.shape[0]
    tokens = x.reshape(B, -1, Q).transpose(0, 2, 1)
    t1 = _layer(tokens, conv1_w, conv1_b, knn1_k, knn1_w, knn1_b, True,
                interpret=interpret)
    t2 = _layer(t1, conv2_w, conv2_b, knn2_k, knn2_w, knn2_b, True,
                interpret=interpret)
    t3 = _layer(t2, conv3_w, conv3_b, knn3_k, knn3_w, knn3_b, False,
                interpret=interpret)
    return t3.transpose(0, 2, 1).reshape(B, -1, H_IMG, W_IMG)


# speedup vs baseline: 2.0256x; 1.1138x over previous
import functools

import jax
import jax.numpy as jnp
from jax.experimental import pallas as pl
from jax.experimental.pallas import tpu as pltpu

H_IMG = 96
W_IMG = 96
Q = H_IMG * W_IMG
NSUB = 8
M = Q // NSUB
K = 9
PAD = 128
QT = 768
NT = Q // QT


def _layer_kernel(tpad_ref, cand_ref, wmat_ref, kw_ref, lw_ref, bias_ref,
                  out_ref, *, cout, relu):
    f32 = jnp.float32
    t = pl.program_id(1)
    q0 = t * QT

    widx = jax.lax.broadcasted_iota(jnp.int32, (QT, 1), 0) % W_IMG
    acc = jnp.zeros((QT, cout), f32)
    for j in range(9):
        dy = j // 3 - 1
        dx = j % 3 - 1
        off = PAD + q0 + dy * W_IMG + dx
        xs = tpad_ref[0, pl.ds(off, QT), :]
        if dx == -1:
            xs = jnp.where(widx > 0, xs, 0.0)
        elif dx == 1:
            xs = jnp.where(widx < W_IMG - 1, xs, 0.0)
        acc = acc + jax.lax.dot_general(
            xs, wmat_ref[j], (((1,), (0,)), ((), ())),
            preferred_element_type=f32)

    tok = tpad_ref[0, pl.ds(PAD + q0, QT), :]
    cand = cand_ref[0]
    q2 = jnp.sum(tok * tok, axis=1, keepdims=True)
    c2 = jnp.sum(cand * cand, axis=1)[None, :]
    dist = q2 - 2.0 * jax.lax.dot_general(
        tok, cand, (((1,), (1,)), ((), ())),
        preferred_element_type=f32) + c2

    it = jax.lax.broadcasted_iota(jnp.int32, (QT, M), 1)
    bits = jax.lax.bitcast_convert_type(jnp.maximum(dist, 0.0), jnp.int32)
    bits = ((bits + 0x8) & ~0xF | (it >> 7)) + 0x00800000
    key = jax.lax.bitcast_convert_type(bits, f32)
    io128 = jax.lax.broadcasted_iota(jnp.int32, (QT, 128), 1).astype(f32)
    BIGD = f32(3e38)
    wsel = jnp.zeros((QT, M), f32)
    NG = M // 128
    for k in range(K):
        mval = key[:, :128]
        for g in range(1, NG):
            mval = jnp.minimum(mval, key[:, g * 128:(g + 1) * 128])
        kmin = jnp.min(mval, axis=1, keepdims=True)
        lstar = jnp.min(jnp.where(mval == kmin, io128, f32(256.0)),
                        axis=1, keepdims=True)
        gstar = jax.lax.bitcast_convert_type(kmin, jnp.int32) & 0xF
        mstar = gstar * 128 + lstar.astype(jnp.int32)
        ohm = it == mstar
        wsel = jnp.where(ohm, kw_ref[0:1, k:k + 1], wsel)
        if k < K - 1:
            key = jnp.where(ohm, BIGD, key)

    agg = jax.lax.dot_general(wsel, cand, (((1,), (0,)), ((), ())),
                              preferred_element_type=f32)
    y = acc + jax.lax.dot_general(agg, lw_ref[...], (((1,), (0,)), ((), ())),
                                  preferred_element_type=f32) + bias_ref[...]
    if relu:
        y = jnp.maximum(y, 0.0)
    out_ref[0] = y


def _layer(tokens, cw, cb, kw, lw, lb, relu, *, interpret=False):
    B, _, cin = tokens.shape
    cout = lw.shape[1]
    tpad = jnp.pad(tokens, ((0, 0), (PAD, PAD), (0, 0)))
    cand = tokens[:, ::NSUB, :]
    wmat = cw.transpose(2, 3, 1, 0).reshape(9, cin, cout)
    bias = (cb + lb)[None, :]
    kw2 = kw[None, :]
    out = pl.pallas_call(
        functools.partial(_layer_kernel, cout=cout, relu=relu),
        grid=(B, NT),
        in_specs=[
            pl.BlockSpec((1, Q + 2 * PAD, cin), lambda b, t: (b, 0, 0)),
            pl.BlockSpec((1, M, cin), lambda b, t: (b, 0, 0)),
            pl.BlockSpec((9, cin, cout), lambda b, t: (0, 0, 0)),
            pl.BlockSpec((1, K), lambda b, t: (0, 0)),
            pl.BlockSpec((cin, cout), lambda b, t: (0, 0)),
            pl.BlockSpec((1, cout), lambda b, t: (0, 0)),
        ],
        out_specs=pl.BlockSpec((1, QT, cout), lambda b, t: (b, t, 0)),
        out_shape=jax.ShapeDtypeStruct((B, Q, cout), jnp.float32),
        compiler_params=pltpu.CompilerParams(
            dimension_semantics=("parallel", "parallel")),
        interpret=interpret,
    )(tpad, cand, wmat, kw2, lw, bias)
    return out


def kernel(x, conv1_w, conv1_b, knn1_k, knn1_w, knn1_b,
           conv2_w, conv2_b, knn2_k, knn2_w, knn2_b,
           conv3_w, conv3_b, knn3_k, knn3_w, knn3_b,
           interpret=False):
    B = x.shape[0]
    tokens = x.reshape(B, -1, Q).transpose(0, 2, 1)
    t1 = _layer(tokens, conv1_w, conv1_b, knn1_k, knn1_w, knn1_b, True,
                interpret=interpret)
    t2 = _layer(t1, conv2_w, conv2_b, knn2_k, knn2_w, knn2_b, True,
                interpret=interpret)
    t3 = _layer(t2, conv3_w, conv3_b, knn3_k, knn3_w, knn3_b, False,
                interpret=interpret)
    return t3.transpose(0, 2, 1).reshape(B, -1, H_IMG, W_IMG)
